# Initial kernel scaffold; baseline (speedup 1.0000x reference)
import jax
import jax.numpy as jnp
from jax.experimental import pallas as pl


def kernel(m, Lz, Ly, Lyz, eps):
    raise NotImplementedError("write your pallas kernel here")


# speedup vs baseline: 9.3028x; 9.3028x over previous
import math

import jax
import jax.numpy as jnp
from jax.experimental import pallas as pl
from jax.experimental.pallas import tpu as pltpu

D_ZB = 64
D_YB = 16
N_YB = 512
N_SB = 128
SB = 64
N_SAMP = 128
JIT_EPS = 1e-4


def _bmm(a, b):
    return jax.lax.dot_general(a, b, (((2,), (1,)), ((0,), (0,))),
                               preferred_element_type=jnp.float32, precision=jax.lax.Precision.HIGHEST)


def _bmm_nt(a, b):
    return jax.lax.dot_general(a, b, (((2,), (2,)), ((0,), (0,))),
                               preferred_element_type=jnp.float32, precision=jax.lax.Precision.HIGHEST)


def _bmm_tn(a, b):
    return jax.lax.dot_general(a, b, (((1,), (1,)), ((0,), (0,))),
                               preferred_element_type=jnp.float32, precision=jax.lax.Precision.HIGHEST)


def _iota2(n):
    ri = jax.lax.broadcasted_iota(jnp.int32, (1, n, n), 1)
    ci = jax.lax.broadcasted_iota(jnp.int32, (1, n, n), 2)
    return ri, ci


def _chol_batched(S, n):
    ri, ci = _iota2(n)
    li = jax.lax.broadcasted_iota(jnp.int32, (1, n), 1)

    def body(k, Sc):
        rowk = jnp.sum(jnp.where(ri == k, Sc, 0.0), axis=1)
        piv = jnp.sum(jnp.where(li == k, rowk, 0.0), axis=1, keepdims=True)
        dinv = jax.lax.rsqrt(piv)
        colk = jnp.where(li >= k, rowk * dinv, 0.0)
        return jnp.where(ci == k, colk[:, :, None],
                         Sc - colk[:, :, None] * colk[:, None, :])

    Sc = jax.lax.fori_loop(0, n, body, S)
    return jnp.where(ri >= ci, Sc, 0.0)


def _tri_inv(L, n):
    base = min(n, 16)
    ri, ci = _iota2(n)
    eye = jnp.where(ri == ci, 1.0, 0.0)
    dg = jnp.sum(jnp.where(ri == ci, L, 0.0), axis=2)
    Lblk = jnp.where((ri // base) == (ci // base), L, 0.0)

    def body(k, X):
        R = eye - _bmm(Lblk, X)
        newrows = R / dg[:, :, None]
        return jnp.where((ri % base) == k, newrows, X)

    X = jax.lax.fori_loop(0, base, body, jnp.zeros_like(L))
    s = base
    while s < n:
        Loff = jnp.where(((ri // (2 * s)) == (ci // (2 * s)))
                         & ((ri // s) != (ci // s)), L, 0.0)
        X = X - _bmm(_bmm(X, Loff), X)
        s *= 2
    return X


def _btr(a):
    eye = jnp.eye(a.shape[1], dtype=jnp.float32)
    return jax.lax.dot_general(a, eye, (((1,), (0,)), ((), ())),
                               preferred_element_type=jnp.float32, precision=jax.lax.Precision.HIGHEST)


def _factor1_kernel(lz_ref, ly_ref, lyz_ref,
                    gzzt_ref, ft_ref, cd_ref, vs_ref, tc_ref):
    f32 = jnp.float32
    n = D_ZB
    ri64 = jax.lax.broadcasted_iota(jnp.int32, (n, n), 0)
    ci64 = jax.lax.broadcasted_iota(jnp.int32, (n, n), 1)
    I64 = jnp.where(ri64 == ci64, 1.0, 0.0).astype(f32)

    Lz_t = jnp.where(ri64 >= ci64, lz_ref[...], 0.0)
    A = jnp.dot(Lz_t, Lz_t.T, preferred_element_type=f32, precision=jax.lax.Precision.HIGHEST) + JIT_EPS * I64
    Hm = jnp.dot(Lz_t.T, Lz_t, preferred_element_type=f32, precision=jax.lax.Precision.HIGHEST) + JIT_EPS * I64
    C2 = _chol_batched(jnp.stack([A, Hm]), n)
    C2inv = _tri_inv(C2, n)
    Gzz = C2[0]
    Gzzinv = C2inv[0]
    CHinv = C2inv[1]
    gzzt_ref[...] = Gzz.T
    ft_ref[...] = jnp.dot(Gzzinv, Lz_t, preferred_element_type=f32, precision=jax.lax.Precision.HIGHEST)
    Rmat = math.sqrt(JIT_EPS) * CHinv.T

    ri16, ci16 = _iota2(D_YB)
    Ly_t = jnp.where(ri16 >= ci16, ly_ref[...], 0.0)
    I16 = jnp.where(ri16 == ci16, 1.0, 0.0).astype(f32)
    Ay = _bmm_nt(Ly_t, Ly_t) + JIT_EPS * I16
    CD = _chol_batched(Ay, D_YB)
    cd_ref[...] = CD
    CDinv = _tri_inv(CD, D_YB)

    Lyzf = lyz_ref[...]
    Usf = jnp.dot(Lyzf, Rmat, preferred_element_type=f32, precision=jax.lax.Precision.HIGHEST)
    Us16 = Usf.reshape(N_YB, D_YB, D_ZB)
    Vs = _bmm(CDinv, Us16).reshape(N_SB, SB, D_ZB)
    vs_ref[...] = Vs

    Gram = _bmm_tn(Vs, Vs)
    ri, ci = _iota2(SB)
    I64b = jnp.where(ri == ci, 1.0, 0.0).astype(f32)

    Tc = jnp.concatenate(
        [jnp.zeros((1, SB, SB), f32), Gram[:-1]], axis=0)
    s = 1
    while s < N_SB:
        Tc = Tc + jnp.concatenate(
            [jnp.zeros((s, SB, SB), f32), Tc[:-s]], axis=0)
        s *= 2
    tc_ref[...] = Tc + I64b


def _factor2_kernel(cd_ref, vs_ref, tc_ref, lyz_ref,
                    bt_ref, et_ref, rt_ref):
    f32 = jnp.float32
    nb = vs_ref.shape[0]
    CD = cd_ref[...]
    Vs = vs_ref[...]
    Tc = tc_ref[...]
    Lyzs = lyz_ref[...].reshape(nb, SB, D_ZB)
    ri, ci = _iota2(SB)
    I64b = jnp.where(ri == ci, 1.0, 0.0).astype(f32)

    CT = _chol_batched(Tc, SB)
    CTinv = _tri_inv(CT, SB)
    P1 = _bmm_nt(CTinv, Vs)
    W = _bmm_tn(CTinv, P1)
    Ssb = I64b + _bmm(Vs, W)
    Db = _chol_batched(Ssb, SB)
    Dbinv = _tri_inv(Db, SB)

    bt_ref[...] = _bmm_nt(Dbinv, W)
    CVs = _bmm(CD, Vs.reshape(4 * nb, D_YB, D_ZB)).reshape(nb, SB, D_ZB)
    CDb = _bmm(CD, Db.reshape(4 * nb, D_YB, SB)).reshape(nb, SB, SB)
    et_ref[...] = _btr(CDb)
    rt_ref[...] = jnp.concatenate([_btr(CVs), _btr(Lyzs)], axis=1)


def _apply_kernel(ez_ref, ey_ref, mz_ref, my_ref, gzzt_ref, ft_ref,
                  bt_ref, et_ref, rt_ref, outz_ref, outy_ref, carry_ref):
    f32 = jnp.float32
    j = pl.program_id(1)

    @pl.when(j == 0)
    def _():
        outz_ref[...] = mz_ref[...] + jnp.dot(
            ez_ref[...], gzzt_ref[...], preferred_element_type=f32, precision=jax.lax.Precision.HIGHEST)
        carry_ref[...] = jnp.zeros((64, SB), f32)

    fz = jnp.dot(ez_ref[...], ft_ref[...], preferred_element_type=f32, precision=jax.lax.Precision.HIGHEST)
    for b in range(8):
        xb = ey_ref[:, b * SB:(b + 1) * SB]
        p = carry_ref[...]
        pfz = jnp.concatenate([p, fz], axis=1)
        yb = (jnp.dot(xb, et_ref[b], preferred_element_type=f32, precision=jax.lax.Precision.HIGHEST)
              + jnp.dot(pfz, rt_ref[b], preferred_element_type=f32, precision=jax.lax.Precision.HIGHEST)
              + my_ref[b:b + 1, :])
        outy_ref[:, b * SB:(b + 1) * SB] = yb
        carry_ref[...] = p + jnp.dot(xb, bt_ref[b], preferred_element_type=f32, precision=jax.lax.Precision.HIGHEST)


@jax.jit
def kernel(m, Lz, Ly, Lyz, eps):
    f32 = jnp.float32
    Lyzf = Lyz.reshape(N_YB * D_YB, D_ZB)

    gzzt, ft, cd, vs, tc = pl.pallas_call(
        _factor1_kernel,
        out_shape=[
            jax.ShapeDtypeStruct((D_ZB, D_ZB), f32),
            jax.ShapeDtypeStruct((D_ZB, D_ZB), f32),
            jax.ShapeDtypeStruct((N_YB, D_YB, D_YB), f32),
            jax.ShapeDtypeStruct((N_SB, SB, D_ZB), f32),
            jax.ShapeDtypeStruct((N_SB, SB, SB), f32),
        ],
        compiler_params=pltpu.CompilerParams(
            vmem_limit_bytes=60 * 1024 * 1024),
    )(Lz, Ly, Lyzf)

    CHK = 16
    NCH = N_SB // CHK
    bt, et, rt = pl.pallas_call(
        _factor2_kernel,
        grid=(NCH,),
        in_specs=[
            pl.BlockSpec((4 * CHK, D_YB, D_YB), lambda j: (j, 0, 0)),
            pl.BlockSpec((CHK, SB, D_ZB), lambda j: (j, 0, 0)),
            pl.BlockSpec((CHK, SB, SB), lambda j: (j, 0, 0)),
            pl.BlockSpec((CHK * SB, D_ZB), lambda j: (j, 0)),
        ],
        out_specs=[
            pl.BlockSpec((CHK, SB, SB), lambda j: (j, 0, 0)),
            pl.BlockSpec((CHK, SB, SB), lambda j: (j, 0, 0)),
            pl.BlockSpec((CHK, 2 * SB, SB), lambda j: (j, 0, 0)),
        ],
        out_shape=[
            jax.ShapeDtypeStruct((N_SB, SB, SB), f32),
            jax.ShapeDtypeStruct((N_SB, SB, SB), f32),
            jax.ShapeDtypeStruct((N_SB, 2 * SB, SB), f32),
        ],
        compiler_params=pltpu.CompilerParams(
            dimension_semantics=("parallel",),
            vmem_limit_bytes=48 * 1024 * 1024),
    )(cd, vs, tc, Lyzf)

    ez = eps[:, :D_ZB]
    ey = eps[:, D_ZB:]
    mz = m[:D_ZB].reshape(1, D_ZB)
    my = m[D_ZB:].reshape(N_SB, SB)

    HS = N_SAMP // 2
    GB = 8
    outz, outy = pl.pallas_call(
        _apply_kernel,
        grid=(2, N_SB // GB),
        in_specs=[
            pl.BlockSpec((HS, D_ZB), lambda i, j: (i, 0)),
            pl.BlockSpec((HS, GB * SB), lambda i, j: (i, j)),
            pl.BlockSpec((1, D_ZB), lambda i, j: (0, 0)),
            pl.BlockSpec((GB, SB), lambda i, j: (j, 0)),
            pl.BlockSpec((D_ZB, D_ZB), lambda i, j: (0, 0)),
            pl.BlockSpec((D_ZB, D_ZB), lambda i, j: (0, 0)),
            pl.BlockSpec((GB, SB, SB), lambda i, j: (j, 0, 0)),
            pl.BlockSpec((GB, SB, SB), lambda i, j: (j, 0, 0)),
            pl.BlockSpec((GB, 2 * SB, SB), lambda i, j: (j, 0, 0)),
        ],
        out_specs=[
            pl.BlockSpec((HS, D_ZB), lambda i, j: (i, 0)),
            pl.BlockSpec((HS, GB * SB), lambda i, j: (i, j)),
        ],
        out_shape=[
            jax.ShapeDtypeStruct((N_SAMP, D_ZB), f32),
            jax.ShapeDtypeStruct((N_SAMP, N_YB * D_YB), f32),
        ],
        scratch_shapes=[pltpu.VMEM((HS, SB), f32)],
        compiler_params=pltpu.CompilerParams(
            dimension_semantics=("parallel", "arbitrary")),
    )(ez, ey, mz, my, gzzt, ft, bt, et, rt)

    return jnp.concatenate([outz, outy], axis=1)
